# baseline (speedup 1.0000x reference)
import functools
import math

import jax
import jax.numpy as jnp
import numpy as np
from jax.experimental import pallas as pl
from jax.experimental.pallas import tpu as pltpu

_B, _TCR, _AA, _FEAT = 8, 2048, 24, 15
_KS = [2, 3, 4, 5, 6, 7]
_FN = [3, 3, 3, 2, 2, 1]
_AF = sum(_FN)
_HID = 10
_K = int(_TCR * 0.05)
_ISQ = 1.0 / math.sqrt(_HID)


def _build_conv_weight(wcs, bcs):
    cols = []
    bias = []
    seg = []
    off = 0
    for (h, fn), wc, bc in zip(zip(_KS, _FN), wcs, bcs):
        w_h = _AA - h + 1
        for o in range(fn):
            seg.append((off, w_h))
            for p in range(w_h):
                col = jnp.zeros((_FEAT, _AA), jnp.float32)
                col = jax.lax.dynamic_update_slice(col, wc[o], (0, p))
                cols.append(col.reshape(-1))
                bias.append(bc[o])
            off += w_h
    wb = jnp.stack(cols, axis=1)
    bcol = jnp.stack(bias)[None, :]
    return wb, bcol, seg


def _attn_kernel(x2_ref, wb_ref, bcol_ref, wq_ref, bq_ref, wk_ref, bk_ref,
                 wv_ref, bv_ref, wa_ref, ba_ref,
                 s_ref, v_ref, cc_ref, *, seg):
    x = x2_ref[0]
    c = jax.lax.dot(x, wb_ref[...])
    c = jax.nn.relu(c + bcol_ref[...])
    feats = [jnp.max(c[:, s:s + w], axis=1, keepdims=True) for s, w in seg]
    feat = jnp.concatenate(feats, axis=1)
    q = jax.lax.dot(feat, wq_ref[...]) + bq_ref[...]
    k = jax.lax.dot(feat, wk_ref[...]) + bk_ref[...]
    v = jax.lax.dot(feat, wv_ref[...]) + bv_ref[...]
    v_ref[0] = v
    rb = 512

    for i in range(_TCR // rb):
        qb = q[i * rb:(i + 1) * rb, :]
        s = jax.lax.dot(qb, k.T) * _ISQ
        s_ref[0, i * rb:(i + 1) * rb, :] = s
        m = jnp.max(s, axis=1, keepdims=True)
        e = jnp.exp(s - m)
        den = jnp.sum(e, axis=1, keepdims=True)
        cc = jax.lax.dot(e / den, v)
        cc_ref[0, i * rb:(i + 1) * rb, :] = jax.lax.dot(cc, wa_ref[...]) + ba_ref[...]


def kernel(x, wc2, bc2, wc3, bc3, wc4, bc4, wc5, bc5, wc6, bc6, wc7, bc7,
           wq, bq, wk, bk, wv, bv, wa, ba, wm, bm):
    wb, bcol, seg = _build_conv_weight(
        [wc2, wc3, wc4, wc5, wc6, wc7], [bc2, bc3, bc4, bc5, bc6, bc7])
    x2 = x.reshape(_B, _TCR, _FEAT * _AA)
    ncol = wb.shape[1]
    full = lambda *shape: pl.BlockSpec(shape, lambda b: (0,) * len(shape))
    s, v, ctx_c = pl.pallas_call(
        functools.partial(_attn_kernel, seg=seg),
        grid=(_B,),
        in_specs=[
            pl.BlockSpec((1, _TCR, _FEAT * _AA), lambda b: (b, 0, 0)),
            full(_FEAT * _AA, ncol), full(1, ncol),
            full(_AF, _HID), full(1, _HID),
            full(_AF, _HID), full(1, _HID),
            full(_AF, _HID), full(1, _HID),
            full(_HID, _AF), full(1, _AF),
        ],
        out_specs=[
            pl.BlockSpec((1, _TCR, _TCR), lambda b: (b, 0, 0)),
            pl.BlockSpec((1, _TCR, _HID), lambda b: (b, 0, 0)),
            pl.BlockSpec((1, _TCR, _AF), lambda b: (b, 0, 0)),
        ],
        out_shape=[
            jax.ShapeDtypeStruct((_B, _TCR, _TCR), jnp.float32),
            jax.ShapeDtypeStruct((_B, _TCR, _HID), jnp.float32),
            jax.ShapeDtypeStruct((_B, _TCR, _AF), jnp.float32),
        ],
    )(x2, wb, bcol,
      wq.T, bq[None, :], wk.T, bk[None, :], wv.T, bv[None, :],
      wa.T, ba[None, :])

    s_g, idx = jax.lax.top_k(s, _K)
    probs = jax.nn.softmax(s_g, axis=-1)
    v_g = v[jnp.arange(_B)[:, None, None], idx]
    ctx = jnp.einsum('btk,btkd->btd', probs, v_g)
    ctx = ctx @ wa.T + ba
    logits = ctx @ wm.T + bm
    out = jnp.sum(logits, axis=1) / _TCR
    return (out, probs[:, None, :, None, :], ctx, ctx_c)


# speedup vs baseline: 11.2952x; 11.2952x over previous
import functools
import math

import jax
import jax.numpy as jnp
from jax import lax
from jax.experimental import pallas as pl
from jax.experimental.pallas import tpu as pltpu
from jax.experimental.pallas import tpu_sc as plsc

_B, _TCR, _AA, _FEAT = 8, 2048, 24, 15
_KS = [2, 3, 4, 5, 6, 7]
_FN = [3, 3, 3, 2, 2, 1]
_AF = sum(_FN)
_HID = 10
_K = int(_TCR * 0.05)
_ISQ = 1.0 / math.sqrt(_HID)
_ROWS = _B * _TCR
_NW = 32
_RPW = _ROWS // _NW
_CAP = 256
_PADP = 112
_NEG = float('-inf')


def _build_conv_weight(wcs, bcs):
    cols = []
    bias = []
    seg = []
    off = 0
    for (h, fn), wc, bc in zip(zip(_KS, _FN), wcs, bcs):
        w_h = _AA - h + 1
        for o in range(fn):
            seg.append((off, w_h))
            for p in range(w_h):
                col = jnp.zeros((_FEAT, _AA), jnp.float32)
                col = jax.lax.dynamic_update_slice(col, wc[o], (0, p))
                cols.append(col.reshape(-1))
                bias.append(bc[o])
            off += w_h
    wb = jnp.stack(cols, axis=1)
    bcol = jnp.stack(bias)[None, :]
    return wb, bcol, seg


def _attn_kernel(x2_ref, wb_ref, bcol_ref, wq_ref, bq_ref, wk_ref, bk_ref,
                 wv_ref, bv_ref, wvr_ref, bvc_ref, wa_ref, ba_ref,
                 s_ref, vt_ref, cc_ref, *, seg):
    x = x2_ref[0]
    c = jax.lax.dot(x, wb_ref[...])
    c = jax.nn.relu(c + bcol_ref[...])
    feats = [jnp.max(c[:, s:s + w], axis=1, keepdims=True) for s, w in seg]
    feat = jnp.concatenate(feats, axis=1)
    q = jax.lax.dot(feat, wq_ref[...]) + bq_ref[...]
    k = jax.lax.dot(feat, wk_ref[...]) + bk_ref[...]
    v = jax.lax.dot(feat, wv_ref[...]) + bv_ref[...]
    vt = jax.lax.dot_general(wvr_ref[...], feat,
                             (((1,), (1,)), ((), ()))) + bvc_ref[...]
    vt_ref[0] = vt
    rb = 512
    for i in range(_TCR // rb):
        qb = q[i * rb:(i + 1) * rb, :]
        s = jax.lax.dot(qb, k.T) * _ISQ
        s_ref[0, i * rb:(i + 1) * rb, :] = s
        m = jnp.max(s, axis=1, keepdims=True)
        e = jnp.exp(s - m)
        den = jnp.sum(e, axis=1, keepdims=True)
        cc = jax.lax.dot(e / den, v)
        cc_ref[0, i * rb:(i + 1) * rb, :] = jax.lax.dot(cc, wa_ref[...]) + ba_ref[...]


def _bitonic_desc(c):
    if len(c) == 1:
        k, p = c[0]
        ks, ps = plsc.sort_key_val(k, p, descending=True)
        return [(ks, ps)]
    half = len(c) // 2
    hi, lo = [], []
    for i in range(half):
        ak, ap = c[i]
        bk, bp = c[i + half]
        m = ak >= bk
        hi.append((jnp.where(m, ak, bk), jnp.where(m, ap, bp)))
        lo.append((jnp.where(m, bk, ak), jnp.where(m, bp, ap)))
    return _bitonic_desc(hi) + _bitonic_desc(lo)


def _merge_desc(a, b):
    brev = [(lax.rev(k, (0,)), lax.rev(p, (0,))) for (k, p) in b[::-1]]
    return _bitonic_desc(a + brev)


def _sort_desc(pairs):
    runs = [_bitonic_desc([x]) for x in pairs]
    while len(runs) > 1:
        nxt = []
        for i in range(0, len(runs) - 1, 2):
            nxt.append(_merge_desc(runs[i], runs[i + 1]))
        if len(runs) % 2:
            nxt.append(runs[-1])
        runs = nxt
    return runs[0]


def _sc_body(s_hbm, vt_hbm, probs_hbm, ctx_hbm,
             vtb, rowb, sv, svi, pstage, cstage):
    wid = lax.axis_index("s") * 2 + lax.axis_index("c")
    r0 = wid * _RPW
    b = r0 // _TCR
    pltpu.sync_copy(vt_hbm.at[pl.ds(b * _HID * _TCR, _HID * _TCR)], vtb)
    zero_i = jnp.zeros((16,), jnp.int32)
    zero_f = jnp.zeros((16,), jnp.float32)
    neg_f = jnp.full((16,), _NEG, jnp.float32)
    cstage[pl.ds(0, 16)] = zero_f
    lane = lax.iota(jnp.int32, 16)

    def row_body(r, carry):
        row = r0 + r
        pltpu.sync_copy(s_hbm.at[pl.ds(row * _TCR, _TCR)], rowb)
        maxima = []
        for g in range(16):
            m = rowb[pl.ds(g * 128, 16)]
            for j in range(1, 8):
                m = jnp.maximum(m, rowb[pl.ds(g * 128 + j * 16, 16)])
            maxima.append((m, zero_i))
        msorted = _sort_desc(maxima)
        tau = msorted[6][0][5]
        for j in range(16):
            sv[pl.ds(j * 16, 16)] = neg_f
            svi[pl.ds(j * 16, 16)] = zero_i
        cnt = zero_i
        for j in range(128):
            v = rowb[pl.ds(j * 16, 16)]
            msk = v >= tau
            pos = cnt + plsc.cumsum(msk.astype(jnp.int32)) - 1
            okm = jnp.logical_and(msk, pos < _CAP)
            plsc.store_scatter(sv, [pos], v, mask=okm)
            plsc.store_scatter(svi, [pos], lane + (j * 16), mask=okm)
            cnt = cnt + plsc.all_reduce_population_count(msk)
        pairs = [(sv[pl.ds(j * 16, 16)], svi[pl.ds(j * 16, 16)])
                 for j in range(16)]
        ss = _sort_desc(pairs)
        mtop = ss[0][0][0]
        es = []
        for j in range(7):
            e = jnp.exp(ss[j][0] - mtop)
            if j == 6:
                e = jnp.where(lane < 6, e, 0.0)
            es.append(e)
        tot = es[0]
        for j in range(1, 7):
            tot = tot + es[j]
        den = jnp.sum(tot)
        ps = [e / den for e in es]
        for j in range(7):
            pstage[pl.ds(j * 16, 16)] = ps[j]
        pltpu.sync_copy(pstage, probs_hbm.at[pl.ds(row * _PADP, _PADP)])
        ctx_vec = zero_f
        for d in range(_HID):
            acc = zero_f
            for j in range(7):
                g = plsc.load_gather(vtb, [ss[j][1] + (d * _TCR)])
                acc = acc + ps[j] * g
            ctx_vec = ctx_vec + jnp.where(lane == d, jnp.sum(acc), 0.0)
        cstage[pl.ds(0, 16)] = ctx_vec
        pltpu.sync_copy(cstage, ctx_hbm.at[pl.ds(row * 16, 16)])
        return carry

    lax.fori_loop(0, _RPW, row_body, 0)


def _final_kernel(ctxr_ref, wa_ref, ba_ref, wm_ref, bm_ref,
                  ctx_ref, out_ref):
    cr = ctxr_ref[...][:, :_HID]
    ctx = jax.lax.dot(cr, wa_ref[...]) + ba_ref[...]
    ctx_ref[...] = ctx
    logits = jax.lax.dot(ctx, wm_ref[...]) + bm_ref[...]
    lg = logits.reshape(_B, _TCR, 2)
    out_ref[...] = jnp.sum(lg, axis=1) * (1.0 / _TCR)


def kernel(x, wc2, bc2, wc3, bc3, wc4, bc4, wc5, bc5, wc6, bc6, wc7, bc7,
           wq, bq, wk, bk, wv, bv, wa, ba, wm, bm):
    wb, bcol, seg = _build_conv_weight(
        [wc2, wc3, wc4, wc5, wc6, wc7], [bc2, bc3, bc4, bc5, bc6, bc7])
    x2 = x.reshape(_B, _TCR, _FEAT * _AA)
    ncol = wb.shape[1]
    full = lambda *shape: pl.BlockSpec(shape, lambda b: (0,) * len(shape))
    s, vt, ctx_c = pl.pallas_call(
        functools.partial(_attn_kernel, seg=seg),
        grid=(_B,),
        in_specs=[
            pl.BlockSpec((1, _TCR, _FEAT * _AA), lambda b: (b, 0, 0)),
            full(_FEAT * _AA, ncol), full(1, ncol),
            full(_AF, _HID), full(1, _HID),
            full(_AF, _HID), full(1, _HID),
            full(_AF, _HID), full(1, _HID),
            full(_HID, _AF), full(_HID, 1),
            full(_HID, _AF), full(1, _AF),
        ],
        out_specs=[
            pl.BlockSpec((1, _TCR, _TCR), lambda b: (b, 0, 0)),
            pl.BlockSpec((1, _HID, _TCR), lambda b: (b, 0, 0)),
            pl.BlockSpec((1, _TCR, _AF), lambda b: (b, 0, 0)),
        ],
        out_shape=[
            jax.ShapeDtypeStruct((_B, _TCR, _TCR), jnp.float32),
            jax.ShapeDtypeStruct((_B, _HID, _TCR), jnp.float32),
            jax.ShapeDtypeStruct((_B, _TCR, _AF), jnp.float32),
        ],
    )(x2, wb, bcol,
      wq.T, bq[None, :], wk.T, bk[None, :], wv.T, bv[None, :], wv, bv[:, None],
      wa.T, ba[None, :])

    s2 = s.reshape(_ROWS * _TCR)
    vtf = vt.reshape(_B * _HID * _TCR)
    mesh = plsc.VectorSubcoreMesh(core_axis_name="c", subcore_axis_name="s")
    probs_pad, ctx_raw = pl.kernel(
        _sc_body,
        out_type=[
            jax.ShapeDtypeStruct((_ROWS * _PADP,), jnp.float32),
            jax.ShapeDtypeStruct((_ROWS * 16,), jnp.float32),
        ],
        mesh=mesh,
        compiler_params=pltpu.CompilerParams(needs_layout_passes=False),
        scratch_types=[
            pltpu.VMEM((_HID * _TCR,), jnp.float32),
            pltpu.VMEM((_TCR,), jnp.float32),
            pltpu.VMEM((_CAP,), jnp.float32),
            pltpu.VMEM((_CAP,), jnp.int32),
            pltpu.VMEM((_PADP,), jnp.float32),
            pltpu.VMEM((16,), jnp.float32),
        ],
    )(s2, vtf)
    ctx_raw = ctx_raw.reshape(_ROWS, 16)

    ctx, out = pl.pallas_call(
        _final_kernel,
        grid=(1,),
        in_specs=[
            full(_ROWS, 16),
            full(_HID, _AF), full(1, _AF),
            full(_AF, 2), full(1, 2),
        ],
        out_specs=[
            full(_ROWS, _AF),
            full(_B, 2),
        ],
        out_shape=[
            jax.ShapeDtypeStruct((_ROWS, _AF), jnp.float32),
            jax.ShapeDtypeStruct((_B, 2), jnp.float32),
        ],
    )(ctx_raw, wa.T, ba[None, :], wm.T, bm[None, :])

    probs_pad = probs_pad.reshape(_ROWS, _PADP)
    probs = probs_pad[:, :_K].reshape(_B, _TCR, _K)[:, None, :, None, :]
    ctx = ctx.reshape(_B, _TCR, _AF)
    return (out, probs, ctx, ctx_c)
